# baseline (speedup 1.0000x reference)
import functools

import jax
import jax.numpy as jnp
from jax import lax
from jax.experimental import pallas as pl
from jax.experimental.pallas import tpu as pltpu
from jax.experimental.pallas import tpu_sc as plsc

_N = 50000
_E = 800000
_G = 256
_D = 64
_NP = 50176
_BN = 1024
_NBLK = _NP // _BN

_NW = 32
_EPW = _E // _NW
_C = 128
_NFULL = _EPW // _C
_TAIL = _EPW - _NFULL * _C


def _prep_body(batch_ref, t_ref, atom_ref, nc_ref, time_ref, edge_ref,
               wh_ref, bh_ref, we_ref, be_ref, ge_ref, bee_ref,
               ta_ref, tg_ref, etab_ref):
    gio = lax.broadcasted_iota(jnp.int32, (_G, _BN), 0)

    def step(i, acc):
        row = batch_ref[pl.ds(i, 1), :]
        cmp = (row == gio).astype(jnp.float32)
        return acc + jnp.sum(cmp, axis=1, keepdims=True)

    counts = lax.fori_loop(0, _NBLK, step,
                           jnp.zeros((_G, 1), jnp.float32))
    counts = jnp.clip(counts.astype(jnp.int32), 0, 1023)

    vio = lax.broadcasted_iota(jnp.int32, (_G, 1024), 1)
    nc_oh = (counts == vio).astype(jnp.float32)
    nc_g = jnp.dot(nc_oh, nc_ref[...],
                   preferred_element_type=jnp.float32)
    t_oh = (t_ref[...] == vio).astype(jnp.float32)
    t_g = jnp.dot(t_oh, time_ref[...],
                  preferred_element_type=jnp.float32)

    wh = wh_ref[...]
    tg = (jnp.dot(nc_g, wh[64:128, :], preferred_element_type=jnp.float32)
          + jnp.dot(t_g, wh[128:192, :], preferred_element_type=jnp.float32)
          + bh_ref[...])
    tg_ref[...] = tg
    ta_ref[...] = jnp.dot(atom_ref[...], wh[0:64, :],
                          preferred_element_type=jnp.float32)

    er = jnp.dot(edge_ref[...], we_ref[...],
                 preferred_element_type=jnp.float32) + be_ref[...]
    er = er * jax.nn.sigmoid(er)
    m = jnp.mean(er, axis=-1, keepdims=True)
    v = jnp.mean((er - m) ** 2, axis=-1, keepdims=True)
    etab_ref[...] = (er - m) / jnp.sqrt(v + 1e-5) * ge_ref[...] + bee_ref[...]


def _prep(batch2d, t_col, atom_table, nc_table, time_pad, edge_table,
          w_h0, b_h0, w_e, b_e, g_e, beta_e):
    return pl.pallas_call(
        _prep_body,
        out_shape=[
            jax.ShapeDtypeStruct((128, 256), jnp.float32),
            jax.ShapeDtypeStruct((_G, 256), jnp.float32),
            jax.ShapeDtypeStruct((8, 128), jnp.float32),
        ],
    )(batch2d, t_col, atom_table, nc_table, time_pad, edge_table,
      w_h0, b_h0, w_e, b_e, g_e, beta_e)


def _node_body(a_ref, b_ref, ta_ref, tg_ref, g_ref, beta_ref, out_ref):
    arow = a_ref[0]
    brow = b_ref[0]
    aio = lax.broadcasted_iota(jnp.int32, (128, _BN), 0)
    bio = lax.broadcasted_iota(jnp.int32, (_G, _BN), 0)
    oh_a = (arow == aio).astype(jnp.float32)
    oh_b = (brow == bio).astype(jnp.float32)
    dn = (((0,), (0,)), ((), ()))
    x = lax.dot_general(oh_a, ta_ref[...], dn,
                        preferred_element_type=jnp.float32)
    x = x + lax.dot_general(oh_b, tg_ref[...], dn,
                            preferred_element_type=jnp.float32)
    x = x * jax.nn.sigmoid(x)
    m = jnp.mean(x, axis=-1, keepdims=True)
    v = jnp.mean((x - m) ** 2, axis=-1, keepdims=True)
    out_ref[...] = (x - m) / jnp.sqrt(v + 1e-5) * g_ref[...] + beta_ref[...]


def _nodes(a3, b3, t_a, t_g, g_h0, beta_h0):
    return pl.pallas_call(
        _node_body,
        grid=(_NBLK,),
        in_specs=[
            pl.BlockSpec((1, 1, _BN), lambda i: (i, 0, 0)),
            pl.BlockSpec((1, 1, _BN), lambda i: (i, 0, 0)),
            pl.BlockSpec((128, 256), lambda i: (0, 0)),
            pl.BlockSpec((_G, 256), lambda i: (0, 0)),
            pl.BlockSpec((1, 256), lambda i: (0, 0)),
            pl.BlockSpec((1, 256), lambda i: (0, 0)),
        ],
        out_specs=pl.BlockSpec((_BN, 256), lambda i: (i, 0)),
        out_shape=jax.ShapeDtypeStruct((_N, 256), jnp.float32),
    )(a3, b3, t_a, t_g, g_h0, beta_h0)


def _edge_body(etab_hbm, e_hbm, out_hbm,
               idx0, idx1, idx2, rows0, rows1, rows2,
               tidx, trows,
               g0, g1, g2, w0, w1, w2, i0, i1, i2):
    idx = (idx0, idx1, idx2)
    rows = (rows0, rows1, rows2)
    gsem = (g0, g1, g2)
    wsem = (w0, w1, w2)
    isem = (i0, i1, i2)

    wid = lax.axis_index("s") * 2 + lax.axis_index("c")
    base = wid * _EPW

    def idx_start(c, b):
        pltpu.make_async_copy(e_hbm.at[pl.ds(base + c * _C, _C)],
                              idx[b], isem[b]).start()

    def idx_wait(b):
        pltpu.make_async_copy(e_hbm.at[pl.ds(0, _C)], idx[b], isem[b]).wait()

    def g_start(b):
        pltpu.make_async_copy(etab_hbm.at[idx[b]], rows[b], gsem[b]).start()

    def g_wait(b):
        pltpu.make_async_copy(etab_hbm.at[idx[b]], rows[b], gsem[b]).wait()

    def w_start(c, b):
        pltpu.make_async_copy(rows[b], out_hbm.at[pl.ds(base + c * _C, _C)],
                              wsem[b]).start()

    def w_wait(b):
        pltpu.make_async_copy(rows[b], out_hbm.at[pl.ds(0, _C)],
                              wsem[b]).wait()

    idx_start(0, 0)
    idx_start(1, 1)
    idx_start(2, 2)
    idx_wait(0)
    g_start(0)
    idx_wait(1)
    g_start(1)

    def body(i, _):
        for b in range(3):
            c = 3 * i + b
            g_wait(b)
            w_start(c, b)
            @pl.when(i < (_NFULL // 3) - 1)
            def _():
                idx_start(c + 3, b)
            nb = (b + 2) % 3
            if b == 0:
                @pl.when(i > 0)
                def _():
                    w_wait(nb)
            else:
                w_wait(nb)
            if b == 0:
                idx_wait(nb)
                g_start(nb)
            else:
                @pl.when(i < (_NFULL // 3) - 1)
                def _():
                    idx_wait(nb)
                    g_start(nb)
        return 0

    lax.fori_loop(0, _NFULL // 3, body, 0)

    pltpu.sync_copy(e_hbm.at[pl.ds(base + _NFULL * _C, _TAIL)], tidx)
    pltpu.make_async_copy(etab_hbm.at[tidx], trows, g0).start()
    pltpu.make_async_copy(etab_hbm.at[tidx], trows, g0).wait()
    pltpu.make_async_copy(
        trows, out_hbm.at[pl.ds(base + _NFULL * _C, _TAIL)], w0).start()
    w_wait(2)
    pltpu.make_async_copy(trows, out_hbm.at[pl.ds(0, _TAIL)], w0).wait()


def _edges(etab, e):
    mesh = plsc.VectorSubcoreMesh(core_axis_name="c", subcore_axis_name="s")
    fn = pl.kernel(
        _edge_body,
        out_type=jax.ShapeDtypeStruct((_E, 128), jnp.float32),
        mesh=mesh,
        scratch_types=[
            pltpu.VMEM((_C,), jnp.int32),
            pltpu.VMEM((_C,), jnp.int32),
            pltpu.VMEM((_C,), jnp.int32),
            pltpu.VMEM((_C, 128), jnp.float32),
            pltpu.VMEM((_C, 128), jnp.float32),
            pltpu.VMEM((_C, 128), jnp.float32),
            pltpu.VMEM((_TAIL,), jnp.int32),
            pltpu.VMEM((_TAIL, 128), jnp.float32),
            pltpu.SemaphoreType.DMA,
            pltpu.SemaphoreType.DMA,
            pltpu.SemaphoreType.DMA,
            pltpu.SemaphoreType.DMA,
            pltpu.SemaphoreType.DMA,
            pltpu.SemaphoreType.DMA,
            pltpu.SemaphoreType.DMA,
            pltpu.SemaphoreType.DMA,
            pltpu.SemaphoreType.DMA,
        ],
    )
    return fn(etab, e)


def kernel(a, e, edge_index, t, batch, atom_table, nc_table, time_table,
           edge_table, W_h0, b_h0, g_h0, beta_h0, W_e, b_e, g_e, beta_e):
    pad = _NP - _N
    a3 = jnp.pad(a, (0, pad)).reshape(_NBLK, 1, _BN)
    batch_p = jnp.pad(batch, (0, pad), constant_values=_G)
    b3 = batch_p.reshape(_NBLK, 1, _BN)
    batch2d = batch_p.reshape(_NBLK, _BN)
    t_col = t.reshape(_G, 1)
    time_pad = jnp.pad(time_table, ((0, 24), (0, 0)))

    t_a, t_g, etab = _prep(
        batch2d, t_col, atom_table, nc_table, time_pad, edge_table,
        W_h0, b_h0.reshape(1, 256), W_e, b_e.reshape(1, 128),
        g_e.reshape(1, 128), beta_e.reshape(1, 128))

    e_embed = _edges(etab, e)
    h0 = _nodes(a3, b3, t_a, t_g,
                g_h0.reshape(1, 256), beta_h0.reshape(1, 256))
    return (h0, edge_index[0], edge_index[1], e_embed)


# speedup vs baseline: 7.2347x; 7.2347x over previous
import functools

import jax
import jax.numpy as jnp
from jax import lax
from jax.experimental import pallas as pl
from jax.experimental.pallas import tpu as pltpu
from jax.experimental.pallas import tpu_sc as plsc

_N = 50000
_E = 800000
_G = 256
_D = 64
_NP = 50176
_BN = 1024
_NBLK = _NP // _BN

_NW = 32
_EPW = _E // _NW
_C = 128
_NFULL = _EPW // _C
_TAIL = _EPW - _NFULL * _C
_R = 64


def _prep_body(batch_ref, t_ref, atom_ref, nc_ref, time_ref, edge_ref,
               wh_ref, bh_ref, we_ref, be_ref, ge_ref, bee_ref,
               ta_ref, tg_ref, etab_ref):
    gio = lax.broadcasted_iota(jnp.int32, (_G, _BN), 0)

    def step(i, acc):
        row = batch_ref[pl.ds(i, 1), :]
        cmp = (row == gio).astype(jnp.float32)
        return acc + jnp.sum(cmp, axis=1, keepdims=True)

    counts = lax.fori_loop(0, _NBLK, step,
                           jnp.zeros((_G, 1), jnp.float32))
    counts = jnp.clip(counts.astype(jnp.int32), 0, 1023)

    vio = lax.broadcasted_iota(jnp.int32, (_G, 1024), 1)
    nc_oh = (counts == vio).astype(jnp.float32)
    nc_g = jnp.dot(nc_oh, nc_ref[...],
                   preferred_element_type=jnp.float32)
    t_oh = (t_ref[...] == vio).astype(jnp.float32)
    t_g = jnp.dot(t_oh, time_ref[...],
                  preferred_element_type=jnp.float32)

    wh = wh_ref[...]
    tg = (jnp.dot(nc_g, wh[64:128, :], preferred_element_type=jnp.float32)
          + jnp.dot(t_g, wh[128:192, :], preferred_element_type=jnp.float32)
          + bh_ref[...])
    tg_ref[...] = tg
    ta_ref[...] = jnp.dot(atom_ref[...], wh[0:64, :],
                          preferred_element_type=jnp.float32)

    er = jnp.dot(edge_ref[...], we_ref[...],
                 preferred_element_type=jnp.float32) + be_ref[...]
    er = er * jax.nn.sigmoid(er)
    m = jnp.mean(er, axis=-1, keepdims=True)
    v = jnp.mean((er - m) ** 2, axis=-1, keepdims=True)
    et = (er - m) / jnp.sqrt(v + 1e-5) * ge_ref[...] + bee_ref[...]
    etab_ref[...] = jnp.broadcast_to(et, (_R, 8, 128)).reshape(_R * 8, 128)


def _prep(batch2d, t_col, atom_table, nc_table, time_pad, edge_table,
          w_h0, b_h0, w_e, b_e, g_e, beta_e):
    return pl.pallas_call(
        _prep_body,
        out_shape=[
            jax.ShapeDtypeStruct((128, 256), jnp.float32),
            jax.ShapeDtypeStruct((_G, 256), jnp.float32),
            jax.ShapeDtypeStruct((_R * 8, 128), jnp.float32),
        ],
    )(batch2d, t_col, atom_table, nc_table, time_pad, edge_table,
      w_h0, b_h0, w_e, b_e, g_e, beta_e)


def _node_body(a_ref, b_ref, ta_ref, tg_ref, g_ref, beta_ref, out_ref):
    arow = a_ref[0]
    brow = b_ref[0]
    aio = lax.broadcasted_iota(jnp.int32, (128, _BN), 0)
    bio = lax.broadcasted_iota(jnp.int32, (_G, _BN), 0)
    oh_a = (arow == aio).astype(jnp.float32)
    oh_b = (brow == bio).astype(jnp.float32)
    dn = (((0,), (0,)), ((), ()))
    x = lax.dot_general(oh_a, ta_ref[...], dn,
                        preferred_element_type=jnp.float32)
    x = x + lax.dot_general(oh_b, tg_ref[...], dn,
                            preferred_element_type=jnp.float32)
    x = x * jax.nn.sigmoid(x)
    m = jnp.mean(x, axis=-1, keepdims=True)
    v = jnp.mean((x - m) ** 2, axis=-1, keepdims=True)
    out_ref[...] = (x - m) / jnp.sqrt(v + 1e-5) * g_ref[...] + beta_ref[...]


def _nodes(a3, b3, t_a, t_g, g_h0, beta_h0):
    return pl.pallas_call(
        _node_body,
        grid=(_NBLK,),
        in_specs=[
            pl.BlockSpec((1, 1, _BN), lambda i: (i, 0, 0)),
            pl.BlockSpec((1, 1, _BN), lambda i: (i, 0, 0)),
            pl.BlockSpec((128, 256), lambda i: (0, 0)),
            pl.BlockSpec((_G, 256), lambda i: (0, 0)),
            pl.BlockSpec((1, 256), lambda i: (0, 0)),
            pl.BlockSpec((1, 256), lambda i: (0, 0)),
        ],
        out_specs=pl.BlockSpec((_BN, 256), lambda i: (i, 0)),
        out_shape=jax.ShapeDtypeStruct((_N, 256), jnp.float32),
    )(a3, b3, t_a, t_g, g_h0, beta_h0)


def _edge_body(etab_hbm, e_hbm, out_hbm,
               idx0, idx1, idx2, rows0, rows1, rows2,
               tidx, trows,
               g0, g1, g2, w0, w1, w2, i0, i1, i2):
    idx = (idx0, idx1, idx2)
    rows = (rows0, rows1, rows2)
    gsem = (g0, g1, g2)
    wsem = (w0, w1, w2)
    isem = (i0, i1, i2)

    wid = lax.axis_index("s") * 2 + lax.axis_index("c")
    base = wid * _EPW

    def idx_start(c, b):
        pltpu.make_async_copy(e_hbm.at[pl.ds(base + c * _C, _C)],
                              idx[b], isem[b]).start()

    def idx_wait(b):
        pltpu.make_async_copy(e_hbm.at[pl.ds(0, _C)], idx[b], isem[b]).wait()

    def add_rep(c, b):
        off = wid * 16 + lax.rem(c, 2) * 8
        for k in range(_C // 16):
            idx[b][pl.ds(16 * k, 16)] = idx[b][pl.ds(16 * k, 16)] + off

    def g_start(b):
        pltpu.make_async_copy(etab_hbm.at[idx[b]], rows[b], gsem[b]).start()

    def g_wait(b):
        pltpu.make_async_copy(etab_hbm.at[idx[b]], rows[b], gsem[b]).wait()

    def w_start(c, b):
        pltpu.make_async_copy(rows[b], out_hbm.at[pl.ds(base + c * _C, _C)],
                              wsem[b]).start()

    def w_wait(b):
        pltpu.make_async_copy(rows[b], out_hbm.at[pl.ds(0, _C)],
                              wsem[b]).wait()

    idx_start(0, 0)
    idx_start(1, 1)
    idx_start(2, 2)
    idx_wait(0)
    add_rep(0, 0)
    g_start(0)
    idx_wait(1)
    add_rep(1, 1)
    g_start(1)

    def body(i, _):
        for b in range(3):
            c = 3 * i + b
            g_wait(b)
            w_start(c, b)
            @pl.when(i < (_NFULL // 3) - 1)
            def _():
                idx_start(c + 3, b)
            nb = (b + 2) % 3
            if b == 0:
                @pl.when(i > 0)
                def _():
                    w_wait(nb)
            else:
                w_wait(nb)
            if b == 0:
                idx_wait(nb)
                add_rep(c + 2, nb)
                g_start(nb)
            else:
                @pl.when(i < (_NFULL // 3) - 1)
                def _():
                    idx_wait(nb)
                    add_rep(c + 2, nb)
                    g_start(nb)
        return 0

    lax.fori_loop(0, _NFULL // 3, body, 0)

    pltpu.sync_copy(e_hbm.at[pl.ds(base + _NFULL * _C, _TAIL)], tidx)
    pltpu.make_async_copy(etab_hbm.at[tidx], trows, g0).start()
    pltpu.make_async_copy(etab_hbm.at[tidx], trows, g0).wait()
    pltpu.make_async_copy(
        trows, out_hbm.at[pl.ds(base + _NFULL * _C, _TAIL)], w0).start()
    w_wait(2)
    pltpu.make_async_copy(trows, out_hbm.at[pl.ds(0, _TAIL)], w0).wait()


def _edges(etab, e):
    mesh = plsc.VectorSubcoreMesh(core_axis_name="c", subcore_axis_name="s")
    fn = pl.kernel(
        _edge_body,
        out_type=jax.ShapeDtypeStruct((_E, 128), jnp.float32),
        mesh=mesh,
        scratch_types=[
            pltpu.VMEM((_C,), jnp.int32),
            pltpu.VMEM((_C,), jnp.int32),
            pltpu.VMEM((_C,), jnp.int32),
            pltpu.VMEM((_C, 128), jnp.float32),
            pltpu.VMEM((_C, 128), jnp.float32),
            pltpu.VMEM((_C, 128), jnp.float32),
            pltpu.VMEM((_TAIL,), jnp.int32),
            pltpu.VMEM((_TAIL, 128), jnp.float32),
            pltpu.SemaphoreType.DMA,
            pltpu.SemaphoreType.DMA,
            pltpu.SemaphoreType.DMA,
            pltpu.SemaphoreType.DMA,
            pltpu.SemaphoreType.DMA,
            pltpu.SemaphoreType.DMA,
            pltpu.SemaphoreType.DMA,
            pltpu.SemaphoreType.DMA,
            pltpu.SemaphoreType.DMA,
        ],
    )
    return fn(etab, e)


def kernel(a, e, edge_index, t, batch, atom_table, nc_table, time_table,
           edge_table, W_h0, b_h0, g_h0, beta_h0, W_e, b_e, g_e, beta_e):
    pad = _NP - _N
    a3 = jnp.pad(a, (0, pad)).reshape(_NBLK, 1, _BN)
    batch_p = jnp.pad(batch, (0, pad), constant_values=_G)
    b3 = batch_p.reshape(_NBLK, 1, _BN)
    batch2d = batch_p.reshape(_NBLK, _BN)
    t_col = t.reshape(_G, 1)
    time_pad = jnp.pad(time_table, ((0, 24), (0, 0)))

    t_a, t_g, etab = _prep(
        batch2d, t_col, atom_table, nc_table, time_pad, edge_table,
        W_h0, b_h0.reshape(1, 256), W_e, b_e.reshape(1, 128),
        g_e.reshape(1, 128), beta_e.reshape(1, 128))

    e_embed = _edges(etab, e)
    h0 = _nodes(a3, b3, t_a, t_g,
                g_h0.reshape(1, 256), beta_h0.reshape(1, 256))
    return (h0, edge_index[0], edge_index[1], e_embed)
